# baseline (speedup 1.0000x reference)
import functools

import jax
import jax.numpy as jnp
from jax import lax
from jax.experimental import pallas as pl
from jax.experimental.pallas import tpu as pltpu
from jax.experimental.pallas import tpu_sc as plsc


EPS = 1e-3

NSC = 2
SC_NC = 2
SC_NS = 16
SC_NW = SC_NC * SC_NS
SC_P = 128


def _stats_kernel(x_ref, sum_ref, sq_ref, cnt_ref, bf_ref):
    b = pl.program_id(0)
    j = pl.program_id(1)

    @pl.when((b == 0) & (j == 0))
    def _():
        sum_ref[...] = jnp.zeros_like(sum_ref)
        sq_ref[...] = jnp.zeros_like(sq_ref)
        cnt_ref[...] = jnp.zeros_like(cnt_ref)
        bf_ref[...] = jnp.zeros_like(bf_ref)

    xb = x_ref[0]
    colsum = jnp.sum(xb, axis=0, keepdims=True)
    maskf = (colsum != 0.0).astype(jnp.float32)
    masked = xb * maskf
    psum = jnp.sum(masked, axis=1, keepdims=True)
    psq = jnp.sum(masked * xb, axis=1, keepdims=True)
    sum_ref[...] = sum_ref[...] + psum
    sq_ref[...] = sq_ref[...] + psq

    cnt = jnp.sum(maskf)
    lanes = jax.lax.broadcasted_iota(jnp.int32, cnt_ref.shape, 1)
    cnt_ref[...] = cnt_ref[...] + jnp.where(lanes == b, cnt, 0.0)

    @pl.when(j == 0)
    def _():
        cols = jax.lax.broadcasted_iota(jnp.int32, bf_ref.shape, 1)
        bf_ref[...] = bf_ref[...] + jnp.where(cols == b, xb[:, 0:1], 0.0)


def _sc_stats_body(x_hbm, sums_hbm, sqs_hbm, cnts_hbm, bfs_hbm,
                   buf0, buf1, accsum, accsq, acccnt, maskbuf, staging,
                   sem0, sem1, *, B0, C, M):
    wid = lax.axis_index("s") * SC_NC + lax.axis_index("c")
    nchunk = 12
    base = wid * (nchunk * SC_P)
    bufs = (buf0, buf1)
    sems = (sem0, sem1)
    NV = SC_P // 16

    def zero_acc(c, carry):
        accsum[c] = jnp.zeros((16,), jnp.float32)
        accsq[c] = jnp.zeros((16,), jnp.float32)
        for kk in range(8):
            staging[c, pl.ds(kk * 16, 16)] = jnp.zeros((16,), jnp.float32)
        return carry

    lax.fori_loop(0, C, zero_acc, 0)
    for bl in range(NSC):
        acccnt[bl] = jnp.zeros((16,), jnp.float32)

    def process(buf, bl):
        def csum(c, accs):
            return tuple(accs[kk] + buf[c, pl.ds(kk * 16, 16)]
                         for kk in range(NV))

        css = lax.fori_loop(
            0, C, csum,
            tuple(jnp.zeros((16,), jnp.float32) for _ in range(NV)))
        for kk in range(NV):
            mk = jnp.where(css[kk] != 0.0, 1.0, 0.0)
            maskbuf[pl.ds(kk * 16, 16)] = mk
            acccnt[bl] = acccnt[bl] + mk

        def chan(c, carry):
            sacc = accsum[c]
            qacc = accsq[c]
            for kk in range(NV):
                xv = buf[c, pl.ds(kk * 16, 16)]
                mv = maskbuf[pl.ds(kk * 16, 16)]
                xm = xv * mv
                sacc = sacc + xm
                qacc = qacc + xm * xv
            accsum[c] = sacc
            accsq[c] = qacc
            return carry

        lax.fori_loop(0, C, chan, 0)

    tasks = [(bl, k) for bl in range(NSC) for k in range(nchunk)]

    def start(t, s):
        bl, k = tasks[t]
        off = base + k * SC_P
        return pltpu.async_copy(
            x_hbm.at[B0 + bl, :, pl.ds(off, SC_P)], bufs[s], sems[s])

    handles = [None, None]
    handles[0] = start(0, 0)
    handles[1] = start(1, 1)

    for t, (bl, k) in enumerate(tasks):
        s = t % 2
        buf = bufs[s]
        handles[s].wait()
        process(buf, bl)

        if k == 0:
            @pl.when(wid == 0)
            def _():
                def bfcopy(c, carry):
                    staging[c, pl.ds(0, 16)] = buf[c, pl.ds(0, 16)]
                    return carry

                lax.fori_loop(0, C, bfcopy, 0)
                pltpu.sync_copy(staging, bfs_hbm.at[bl])

        nt = t + 2
        if nt < len(tasks):
            handles[s] = start(nt, s)

    tail = SC_NW * nchunk * SC_P

    @pl.when(wid < 8)
    def _():
        for bl in range(NSC):
            off = tail + wid * 128
            pltpu.async_copy(
                x_hbm.at[B0 + bl, :, pl.ds(off, 128)], buf0, sem0).wait()
            process(buf0, bl)

    def exp_sum(c, carry):
        staging[c, pl.ds(0, 16)] = accsum[c]
        return carry

    lax.fori_loop(0, C, exp_sum, 0)
    pltpu.sync_copy(staging, sums_hbm.at[wid])

    def exp_sq(c, carry):
        staging[c, pl.ds(0, 16)] = accsq[c]
        return carry

    lax.fori_loop(0, C, exp_sq, 0)
    pltpu.sync_copy(staging, sqs_hbm.at[wid])

    for r in range(8):
        if r < NSC:
            staging[r, pl.ds(0, 16)] = acccnt[r]
        else:
            staging[r, pl.ds(0, 16)] = jnp.zeros((16,), jnp.float32)
    pltpu.sync_copy(staging.at[0:8, :], cnts_hbm.at[wid])


def _scale_kernel(x_hbm, sum_ref, sq_ref, cnt_ref, bf_ref,
                  scs_ref, scq_ref, scc_ref, scb_ref, o_hbm,
                  inb, outb, insem, outsem, *, M, NT, BMk, NB, JK, NCH, B0):
    i = pl.program_id(0)

    def in_copy(step, s):
        bb = step // JK
        oo = (step % JK) * BMk
        return pltpu.make_async_copy(
            x_hbm.at[bb, :, pl.ds(oo, BMk)], inb.at[s], insem.at[s])

    def out_copy(step, s):
        bb = step // JK
        oo = (step % JK) * BMk
        return pltpu.make_async_copy(
            outb.at[s], o_hbm.at[bb, :, pl.ds(oo, BMk)], outsem.at[s])

    @pl.when(i == 0)
    def _():
        for s in range(NB):
            in_copy(s, s).start()

    lanes8 = jax.lax.broadcasted_iota(jnp.int32, (1, 8), 1)
    NW = scs_ref.shape[0]
    sc_s = scs_ref[0]
    sc_q = scq_ref[0]
    sc_c = scc_ref[0]
    for w in range(1, NW):
        sc_s = sc_s + scs_ref[w]
        sc_q = sc_q + scq_ref[w]
        sc_c = sc_c + scc_ref[w]

    nrow = cnt_ref[0:1, 0:8]
    for bl in range(NSC):
        nrow = nrow + jnp.where(
            lanes8 == B0 + bl, jnp.sum(sc_c[bl:bl + 1, :]), 0.0)

    cols8 = jax.lax.broadcasted_iota(jnp.int32, bf_ref.shape, 1)
    bf = bf_ref[...]
    for bl in range(NSC):
        bf = bf + jnp.where(cols8 == B0 + bl, scb_ref[bl][:, 0:1], 0.0)

    padrow = jnp.float32(M) - nrow
    s_tot = (sum_ref[:, 0:1]
             + jnp.sum(sc_s, axis=1, keepdims=True)
             + jnp.sum(bf * padrow, axis=1, keepdims=True))
    q_tot = (sq_ref[:, 0:1]
             + jnp.sum(sc_q, axis=1, keepdims=True)
             + jnp.sum(bf * bf * padrow, axis=1, keepdims=True))
    mean = s_tot * (1.0 / NT)
    var = q_tot * (1.0 / NT) - mean * mean
    inv = jax.lax.rsqrt(var + EPS)

    for s in range(NB):
        step = i * NB + s
        bb = step // JK
        jj = step % JK

        in_copy(step, s).wait()

        @pl.when(step >= NB)
        def _():
            out_copy(step - NB, s).wait()

        xb = inb[s]
        colsum = jnp.sum(xb, axis=0, keepdims=True)
        wm = colsum != 0.0

        nb_ = jnp.sum(jnp.where(lanes8 == bb, nrow, 0.0))
        lanes = jax.lax.broadcasted_iota(jnp.int32, wm.shape, 1)
        wm = wm | ((jj == 0) & (nb_ < M) & (lanes == 0))

        outb[s] = jnp.where(wm, xb * inv, xb)
        out_copy(step, s).start()

        nstep = step + NB

        @pl.when(nstep < NCH)
        def _():
            in_copy(nstep, s).start()

    @pl.when(i == (NCH // NB) - 1)
    def _():
        for s in range(NB):
            out_copy(NCH - NB + s, s).wait()


def kernel(x):
    B, C, W, H = x.shape
    M = W * H
    B0 = B - NSC
    BM = 12544
    J = M // BM
    xr = x.reshape(B, C, M)

    x_spec = pl.BlockSpec((1, C, BM), lambda b, j: (b, 0, j))

    def const_spec(shape):
        return pl.BlockSpec(shape, lambda b, j: (0,) * len(shape))

    stats_shapes = [
        jax.ShapeDtypeStruct((C, 128), jnp.float32),
        jax.ShapeDtypeStruct((C, 128), jnp.float32),
        jax.ShapeDtypeStruct((1, 128), jnp.float32),
        jax.ShapeDtypeStruct((C, 8), jnp.float32),
    ]
    sums, sqs, cnts, bf = pl.pallas_call(
        _stats_kernel,
        grid=(B0, J),
        in_specs=[x_spec],
        out_specs=[const_spec(s.shape) for s in stats_shapes],
        out_shape=stats_shapes,
    )(xr)

    mesh = plsc.VectorSubcoreMesh(core_axis_name="c", subcore_axis_name="s")
    sc_stats = functools.partial(
        pl.kernel,
        mesh=mesh,
        out_type=[
            jax.ShapeDtypeStruct((SC_NW, C, 128), jnp.float32),
            jax.ShapeDtypeStruct((SC_NW, C, 128), jnp.float32),
            jax.ShapeDtypeStruct((SC_NW, 8, 128), jnp.float32),
            jax.ShapeDtypeStruct((NSC, C, 128), jnp.float32),
        ],
        scratch_types=[
            pltpu.VMEM((C, SC_P), jnp.float32),
            pltpu.VMEM((C, SC_P), jnp.float32),
            pltpu.VMEM((C, 16), jnp.float32),
            pltpu.VMEM((C, 16), jnp.float32),
            pltpu.VMEM((NSC, 16), jnp.float32),
            pltpu.VMEM((SC_P,), jnp.float32),
            pltpu.VMEM((C, 128), jnp.float32),
            pltpu.SemaphoreType.DMA,
            pltpu.SemaphoreType.DMA,
        ],
    )(functools.partial(_sc_stats_body, B0=B0, C=C, M=M))
    scs, scq, scc, scb = sc_stats(xr)

    BMk = 1792
    JK = M // BMk
    NCH = B * JK
    NB = 7
    c_spec = pl.BlockSpec(memory_space=pl.ANY)

    def cs(shape):
        return pl.BlockSpec(shape, lambda i: (0,) * len(shape))

    out = pl.pallas_call(
        functools.partial(_scale_kernel, M=M, NT=float(B * M),
                          BMk=BMk, NB=NB, JK=JK, NCH=NCH, B0=B0),
        grid=(NCH // NB,),
        in_specs=[
            c_spec,
            cs((C, 128)),
            cs((C, 128)),
            cs((1, 128)),
            cs((C, 8)),
            cs((SC_NW, C, 128)),
            cs((SC_NW, C, 128)),
            cs((SC_NW, 8, 128)),
            cs((NSC, C, 128)),
        ],
        out_specs=c_spec,
        out_shape=jax.ShapeDtypeStruct((B, C, M), jnp.float32),
        scratch_shapes=[
            pltpu.VMEM((NB, C, BMk), jnp.float32),
            pltpu.VMEM((NB, C, BMk), jnp.float32),
            pltpu.SemaphoreType.DMA((NB,)),
            pltpu.SemaphoreType.DMA((NB,)),
        ],
    )(xr, sums, sqs, cnts, bf, scs, scq, scc, scb)

    return out.reshape(B, C, W, H)


# speedup vs baseline: 1.0000x; 1.0000x over previous
import functools

import jax
import jax.numpy as jnp
from jax import lax
from jax.experimental import pallas as pl
from jax.experimental.pallas import tpu as pltpu
from jax.experimental.pallas import tpu_sc as plsc


EPS = 1e-3

NSC = 2
SC_NC = 2
SC_NS = 16
SC_NW = SC_NC * SC_NS
SC_P = 128


def _stats_kernel(x_ref, sum_ref, sq_ref, cnt_ref, bf_ref):
    b = pl.program_id(0)
    j = pl.program_id(1)

    @pl.when((b == 0) & (j == 0))
    def _():
        sum_ref[...] = jnp.zeros_like(sum_ref)
        sq_ref[...] = jnp.zeros_like(sq_ref)
        cnt_ref[...] = jnp.zeros_like(cnt_ref)
        bf_ref[...] = jnp.zeros_like(bf_ref)

    xb = x_ref[0]
    colsum = jnp.sum(xb, axis=0, keepdims=True)
    maskf = (colsum != 0.0).astype(jnp.float32)
    masked = xb * maskf
    psum = jnp.sum(masked, axis=1, keepdims=True)
    psq = jnp.sum(masked * xb, axis=1, keepdims=True)
    sum_ref[...] = sum_ref[...] + psum
    sq_ref[...] = sq_ref[...] + psq

    cnt = jnp.sum(maskf)
    lanes = jax.lax.broadcasted_iota(jnp.int32, cnt_ref.shape, 1)
    cnt_ref[...] = cnt_ref[...] + jnp.where(lanes == b, cnt, 0.0)

    @pl.when(j == 0)
    def _():
        cols = jax.lax.broadcasted_iota(jnp.int32, bf_ref.shape, 1)
        bf_ref[...] = bf_ref[...] + jnp.where(cols == b, xb[:, 0:1], 0.0)


def _sc_stats_body(x_hbm, sums_hbm, sqs_hbm, cnts_hbm, bfs_hbm,
                   buf0, buf1, accsum, accsq, acccnt, maskbuf, staging,
                   sem0, sem1, *, B0, C, M):
    wid = lax.axis_index("s") * SC_NC + lax.axis_index("c")
    nchunk = 12
    base = wid * (nchunk * SC_P)
    bufs = (buf0, buf1)
    sems = (sem0, sem1)
    NV = SC_P // 16

    def zero_acc(c, carry):
        accsum[c] = jnp.zeros((16,), jnp.float32)
        accsq[c] = jnp.zeros((16,), jnp.float32)
        for kk in range(8):
            staging[c, pl.ds(kk * 16, 16)] = jnp.zeros((16,), jnp.float32)
        return carry

    lax.fori_loop(0, C, zero_acc, 0)
    for bl in range(NSC):
        acccnt[bl] = jnp.zeros((16,), jnp.float32)

    def process(buf, bl):
        def csum(c, accs):
            return tuple(accs[kk] + buf[c, pl.ds(kk * 16, 16)]
                         for kk in range(NV))

        css = lax.fori_loop(
            0, C, csum,
            tuple(jnp.zeros((16,), jnp.float32) for _ in range(NV)))
        for kk in range(NV):
            mk = jnp.where(css[kk] != 0.0, 1.0, 0.0)
            maskbuf[pl.ds(kk * 16, 16)] = mk
            acccnt[bl] = acccnt[bl] + mk

        def chan(c, carry):
            sacc = accsum[c]
            qacc = accsq[c]
            for kk in range(NV):
                xv = buf[c, pl.ds(kk * 16, 16)]
                mv = maskbuf[pl.ds(kk * 16, 16)]
                xm = xv * mv
                sacc = sacc + xm
                qacc = qacc + xm * xv
            accsum[c] = sacc
            accsq[c] = qacc
            return carry

        lax.fori_loop(0, C, chan, 0)

    tasks = [(bl, k) for bl in range(NSC) for k in range(nchunk)]

    def start(t, s):
        bl, k = tasks[t]
        off = base + k * SC_P
        return pltpu.async_copy(
            x_hbm.at[B0 + bl, :, pl.ds(off, SC_P)], bufs[s], sems[s])

    handles = [None, None]
    handles[0] = start(0, 0)
    handles[1] = start(1, 1)

    for t, (bl, k) in enumerate(tasks):
        s = t % 2
        buf = bufs[s]
        handles[s].wait()
        process(buf, bl)

        if k == 0:
            @pl.when(wid == 0)
            def _():
                def bfcopy(c, carry):
                    staging[c, pl.ds(0, 16)] = buf[c, pl.ds(0, 16)]
                    return carry

                lax.fori_loop(0, C, bfcopy, 0)
                pltpu.sync_copy(staging, bfs_hbm.at[bl])

        nt = t + 2
        if nt < len(tasks):
            handles[s] = start(nt, s)

    tail = SC_NW * nchunk * SC_P

    @pl.when(wid < 8)
    def _():
        for bl in range(NSC):
            off = tail + wid * 128
            pltpu.async_copy(
                x_hbm.at[B0 + bl, :, pl.ds(off, 128)], buf0, sem0).wait()
            process(buf0, bl)

    def exp_sum(c, carry):
        staging[c, pl.ds(0, 16)] = accsum[c]
        return carry

    lax.fori_loop(0, C, exp_sum, 0)
    pltpu.sync_copy(staging, sums_hbm.at[wid])

    def exp_sq(c, carry):
        staging[c, pl.ds(0, 16)] = accsq[c]
        return carry

    lax.fori_loop(0, C, exp_sq, 0)
    pltpu.sync_copy(staging, sqs_hbm.at[wid])

    for r in range(8):
        if r < NSC:
            staging[r, pl.ds(0, 16)] = acccnt[r]
        else:
            staging[r, pl.ds(0, 16)] = jnp.zeros((16,), jnp.float32)
    pltpu.sync_copy(staging.at[0:8, :], cnts_hbm.at[wid])


def _scale_kernel(x_hbm, sum_ref, sq_ref, cnt_ref, bf_ref,
                  scs_ref, scq_ref, scc_ref, scb_ref, o_hbm,
                  inb, outb, insem, outsem, *, M, NT, BMk, NB, JK, NCH, B0):
    i = pl.program_id(0)

    def in_copy(step, s):
        bb = step // JK
        oo = (step % JK) * BMk
        return pltpu.make_async_copy(
            x_hbm.at[bb, :, pl.ds(oo, BMk)], inb.at[s], insem.at[s])

    def out_copy(step, s):
        bb = step // JK
        oo = (step % JK) * BMk
        return pltpu.make_async_copy(
            outb.at[s], o_hbm.at[bb, :, pl.ds(oo, BMk)], outsem.at[s])

    @pl.when(i == 0)
    def _():
        for s in range(NB):
            in_copy(s, s).start()

    lanes8 = jax.lax.broadcasted_iota(jnp.int32, (1, 8), 1)
    NW = scs_ref.shape[0]
    sc_s = scs_ref[0]
    sc_q = scq_ref[0]
    sc_c = scc_ref[0]
    for w in range(1, NW):
        sc_s = sc_s + scs_ref[w]
        sc_q = sc_q + scq_ref[w]
        sc_c = sc_c + scc_ref[w]

    nrow = cnt_ref[0:1, 0:8]
    for bl in range(NSC):
        nrow = nrow + jnp.where(
            lanes8 == B0 + bl, jnp.sum(sc_c[bl:bl + 1, :]), 0.0)

    cols8 = jax.lax.broadcasted_iota(jnp.int32, bf_ref.shape, 1)
    bf = bf_ref[...]
    for bl in range(NSC):
        bf = bf + jnp.where(cols8 == B0 + bl, scb_ref[bl][:, 0:1], 0.0)

    padrow = jnp.float32(M) - nrow
    s_tot = (sum_ref[:, 0:1]
             + jnp.sum(sc_s, axis=1, keepdims=True)
             + jnp.sum(bf * padrow, axis=1, keepdims=True))
    q_tot = (sq_ref[:, 0:1]
             + jnp.sum(sc_q, axis=1, keepdims=True)
             + jnp.sum(bf * bf * padrow, axis=1, keepdims=True))
    mean = s_tot * (1.0 / NT)
    var = q_tot * (1.0 / NT) - mean * mean
    inv = jax.lax.rsqrt(var + EPS)

    for s in range(NB):
        step = i * NB + s
        bb = step // JK
        jj = step % JK

        in_copy(step, s).wait()

        @pl.when(step >= NB)
        def _():
            out_copy(step - NB, s).wait()

        xb = inb[s]
        colsum = jnp.sum(xb, axis=0, keepdims=True)
        wm = colsum != 0.0

        nb_ = jnp.sum(jnp.where(lanes8 == bb, nrow, 0.0))
        lanes = jax.lax.broadcasted_iota(jnp.int32, wm.shape, 1)
        wm = wm | ((jj == 0) & (nb_ < M) & (lanes == 0))

        outb[s] = jnp.where(wm, xb * inv, xb)
        out_copy(step, s).start()

        nstep = step + NB

        @pl.when(nstep < NCH)
        def _():
            in_copy(nstep, s).start()

    @pl.when(i == (NCH // NB) - 1)
    def _():
        for s in range(NB):
            out_copy(NCH - NB + s, s).wait()


def kernel(x):
    B, C, W, H = x.shape
    M = W * H
    B0 = B - NSC
    BM = 12544
    J = M // BM
    xr = x.reshape(B, C, M)

    x_spec = pl.BlockSpec((1, C, BM), lambda b, j: (b, 0, j))

    def const_spec(shape):
        return pl.BlockSpec(shape, lambda b, j: (0,) * len(shape))

    stats_shapes = [
        jax.ShapeDtypeStruct((C, 128), jnp.float32),
        jax.ShapeDtypeStruct((C, 128), jnp.float32),
        jax.ShapeDtypeStruct((1, 128), jnp.float32),
        jax.ShapeDtypeStruct((C, 8), jnp.float32),
    ]
    mesh = plsc.VectorSubcoreMesh(core_axis_name="c", subcore_axis_name="s")
    sc_stats = functools.partial(
        pl.kernel,
        mesh=mesh,
        out_type=[
            jax.ShapeDtypeStruct((SC_NW, C, 128), jnp.float32),
            jax.ShapeDtypeStruct((SC_NW, C, 128), jnp.float32),
            jax.ShapeDtypeStruct((SC_NW, 8, 128), jnp.float32),
            jax.ShapeDtypeStruct((NSC, C, 128), jnp.float32),
        ],
        scratch_types=[
            pltpu.VMEM((C, SC_P), jnp.float32),
            pltpu.VMEM((C, SC_P), jnp.float32),
            pltpu.VMEM((C, 16), jnp.float32),
            pltpu.VMEM((C, 16), jnp.float32),
            pltpu.VMEM((NSC, 16), jnp.float32),
            pltpu.VMEM((SC_P,), jnp.float32),
            pltpu.VMEM((C, 128), jnp.float32),
            pltpu.SemaphoreType.DMA,
            pltpu.SemaphoreType.DMA,
        ],
    )(functools.partial(_sc_stats_body, B0=B0, C=C, M=M))
    scs, scq, scc, scb = sc_stats(xr)

    sums, sqs, cnts, bf = pl.pallas_call(
        _stats_kernel,
        grid=(B0, J),
        in_specs=[x_spec],
        out_specs=[const_spec(s.shape) for s in stats_shapes],
        out_shape=stats_shapes,
    )(xr)

    BMk = 1792
    JK = M // BMk
    NCH = B * JK
    NB = 7
    c_spec = pl.BlockSpec(memory_space=pl.ANY)

    def cs(shape):
        return pl.BlockSpec(shape, lambda i: (0,) * len(shape))

    out = pl.pallas_call(
        functools.partial(_scale_kernel, M=M, NT=float(B * M),
                          BMk=BMk, NB=NB, JK=JK, NCH=NCH, B0=B0),
        grid=(NCH // NB,),
        in_specs=[
            c_spec,
            cs((C, 128)),
            cs((C, 128)),
            cs((1, 128)),
            cs((C, 8)),
            cs((SC_NW, C, 128)),
            cs((SC_NW, C, 128)),
            cs((SC_NW, 8, 128)),
            cs((NSC, C, 128)),
        ],
        out_specs=c_spec,
        out_shape=jax.ShapeDtypeStruct((B, C, M), jnp.float32),
        scratch_shapes=[
            pltpu.VMEM((NB, C, BMk), jnp.float32),
            pltpu.VMEM((NB, C, BMk), jnp.float32),
            pltpu.SemaphoreType.DMA((NB,)),
            pltpu.SemaphoreType.DMA((NB,)),
        ],
    )(xr, sums, sqs, cnts, bf, scs, scq, scc, scb)

    return out.reshape(B, C, W, H)


# speedup vs baseline: 1.0499x; 1.0498x over previous
import functools

import jax
import jax.numpy as jnp
from jax.experimental import pallas as pl
from jax.experimental.pallas import tpu as pltpu


EPS = 1e-3


def _stats_kernel(x_hbm, sum_ref, sq_ref, cnt_ref, bf_ref,
                  inb, insem, *, BMk, NB, JK, NCH):
    i = pl.program_id(0)

    def in_copy(step, s):
        bb = step // JK
        oo = (step % JK) * BMk
        return pltpu.make_async_copy(
            x_hbm.at[bb, :, pl.ds(oo, BMk)], inb.at[s], insem.at[s])

    @pl.when(i == 0)
    def _():
        sum_ref[...] = jnp.zeros_like(sum_ref)
        sq_ref[...] = jnp.zeros_like(sq_ref)
        cnt_ref[...] = jnp.zeros_like(cnt_ref)
        bf_ref[...] = jnp.zeros_like(bf_ref)
        for s in range(NB):
            in_copy(s, s).start()

    for s in range(NB):
        step = i * NB + s
        bb = step // JK
        jj = step % JK

        in_copy(step, s).wait()
        xb = inb[s]

        colsum = jnp.sum(xb, axis=0, keepdims=True)
        maskf = (colsum != 0.0).astype(jnp.float32)
        masked = xb * maskf
        sum_ref[...] = sum_ref[...] + jnp.sum(masked, axis=1, keepdims=True)
        sq_ref[...] = sq_ref[...] + jnp.sum(masked * xb, axis=1,
                                            keepdims=True)

        cnt = jnp.sum(maskf)
        lanes = jax.lax.broadcasted_iota(jnp.int32, cnt_ref.shape, 1)
        cnt_ref[...] = cnt_ref[...] + jnp.where(lanes == bb, cnt, 0.0)

        @pl.when(jj == 0)
        def _():
            cols = jax.lax.broadcasted_iota(jnp.int32, bf_ref.shape, 1)
            bf_ref[...] = bf_ref[...] + jnp.where(cols == bb,
                                                  xb[:, 0:1], 0.0)

        nstep = step + NB

        @pl.when(nstep < NCH)
        def _():
            in_copy(nstep, s).start()


def _scale_kernel(x_hbm, sum_ref, sq_ref, cnt_ref, bf_ref, o_hbm,
                  inb, outb, insem, outsem, *, M, NT, BMk, NB, JK, NCH):
    i = pl.program_id(0)

    def in_copy(step, s):
        bb = step // JK
        oo = (step % JK) * BMk
        return pltpu.make_async_copy(
            x_hbm.at[bb, :, pl.ds(oo, BMk)], inb.at[s], insem.at[s])

    def out_copy(step, s):
        bb = step // JK
        oo = (step % JK) * BMk
        return pltpu.make_async_copy(
            outb.at[s], o_hbm.at[bb, :, pl.ds(oo, BMk)], outsem.at[s])

    @pl.when(i == 0)
    def _():
        for s in range(NB):
            in_copy(s, s).start()

    nrow = cnt_ref[0:1, 0:8]
    padrow = jnp.float32(M) - nrow
    bf = bf_ref[...]
    s_tot = sum_ref[:, 0:1] + jnp.sum(bf * padrow, axis=1, keepdims=True)
    q_tot = sq_ref[:, 0:1] + jnp.sum(bf * bf * padrow, axis=1, keepdims=True)
    mean = s_tot * (1.0 / NT)
    var = q_tot * (1.0 / NT) - mean * mean
    inv = jax.lax.rsqrt(var + EPS)
    lanes8 = jax.lax.broadcasted_iota(jnp.int32, (1, 8), 1)

    for s in range(NB):
        step = i * NB + s
        bb = step // JK
        jj = step % JK

        in_copy(step, s).wait()

        @pl.when(step >= NB)
        def _():
            out_copy(step - NB, s).wait()

        xb = inb[s]
        colsum = jnp.sum(xb, axis=0, keepdims=True)
        wm = colsum != 0.0

        nb_ = jnp.sum(jnp.where(lanes8 == bb, nrow, 0.0))
        lanes = jax.lax.broadcasted_iota(jnp.int32, wm.shape, 1)
        wm = wm | ((jj == 0) & (nb_ < M) & (lanes == 0))

        outb[s] = jnp.where(wm, xb * inv, xb)
        out_copy(step, s).start()

        nstep = step + NB

        @pl.when(nstep < NCH)
        def _():
            in_copy(nstep, s).start()

    @pl.when(i == (NCH // NB) - 1)
    def _():
        for s in range(NB):
            out_copy(NCH - NB + s, s).wait()


def kernel(x):
    B, C, W, H = x.shape
    M = W * H
    BM = 12544
    J = M // BM
    xr = x.reshape(B, C, M)

    stats_shapes = [
        jax.ShapeDtypeStruct((C, 128), jnp.float32),
        jax.ShapeDtypeStruct((C, 128), jnp.float32),
        jax.ShapeDtypeStruct((1, 128), jnp.float32),
        jax.ShapeDtypeStruct((C, 8), jnp.float32),
    ]
    BMs = 3584
    JKs = M // BMs
    NCHs = B * JKs
    NBs = 7

    def cs1(shape):
        return pl.BlockSpec(shape, lambda i: (0,) * len(shape))

    sums, sqs, cnts, bf = pl.pallas_call(
        functools.partial(_stats_kernel, BMk=BMs, NB=NBs, JK=JKs, NCH=NCHs),
        grid=(NCHs // NBs,),
        in_specs=[pl.BlockSpec(memory_space=pl.ANY)],
        out_specs=[cs1(s.shape) for s in stats_shapes],
        out_shape=stats_shapes,
        scratch_shapes=[
            pltpu.VMEM((NBs, C, BMs), jnp.float32),
            pltpu.SemaphoreType.DMA((NBs,)),
        ],
    )(xr)

    BMk = 1792
    JK = M // BMk
    NCH = B * JK
    NB = 7
    c_spec = pl.BlockSpec(memory_space=pl.ANY)

    def cs(shape):
        return pl.BlockSpec(shape, lambda i: (0,) * len(shape))

    out = pl.pallas_call(
        functools.partial(_scale_kernel, M=M, NT=float(B * M),
                          BMk=BMk, NB=NB, JK=JK, NCH=NCH),
        grid=(NCH // NB,),
        in_specs=[
            c_spec,
            cs((C, 128)),
            cs((C, 128)),
            cs((1, 128)),
            cs((C, 8)),
        ],
        out_specs=c_spec,
        out_shape=jax.ShapeDtypeStruct((B, C, M), jnp.float32),
        scratch_shapes=[
            pltpu.VMEM((NB, C, BMk), jnp.float32),
            pltpu.VMEM((NB, C, BMk), jnp.float32),
            pltpu.SemaphoreType.DMA((NB,)),
            pltpu.SemaphoreType.DMA((NB,)),
        ],
    )(xr, sums, sqs, cnts, bf)

    return out.reshape(B, C, W, H)


# speedup vs baseline: 1.0499x; 1.0000x over previous
import functools

import jax
import jax.numpy as jnp
from jax.experimental import pallas as pl
from jax.experimental.pallas import tpu as pltpu


EPS = 1e-3


def _stats_kernel(x_hbm, sum_ref, sq_ref, cnt_ref, bf_ref,
                  inb, insem, *, BMk, NB, JK, NCH):
    i = pl.program_id(0)

    def in_copy(step, s):
        bb = step // JK
        oo = (step % JK) * BMk
        return pltpu.make_async_copy(
            x_hbm.at[bb, :, pl.ds(oo, BMk)], inb.at[s], insem.at[s])

    @pl.when(i == 0)
    def _():
        sum_ref[...] = jnp.zeros_like(sum_ref)
        sq_ref[...] = jnp.zeros_like(sq_ref)
        cnt_ref[...] = jnp.zeros_like(cnt_ref)
        bf_ref[...] = jnp.zeros_like(bf_ref)
        for s in range(NB):
            in_copy(s, s).start()

    for s in range(NB):
        step = i * NB + s
        bb = step // JK
        jj = step % JK

        in_copy(step, s).wait()
        xb = inb[s]

        colsum = jnp.sum(xb, axis=0, keepdims=True)
        maskf = (colsum != 0.0).astype(jnp.float32)
        masked = xb * maskf
        sum_ref[...] = sum_ref[...] + jnp.sum(masked, axis=1, keepdims=True)
        sq_ref[...] = sq_ref[...] + jnp.sum(masked * xb, axis=1,
                                            keepdims=True)

        cnt = jnp.sum(maskf)
        lanes = jax.lax.broadcasted_iota(jnp.int32, cnt_ref.shape, 1)
        cnt_ref[...] = cnt_ref[...] + jnp.where(lanes == bb, cnt, 0.0)

        @pl.when(jj == 0)
        def _():
            cols = jax.lax.broadcasted_iota(jnp.int32, bf_ref.shape, 1)
            bf_ref[...] = bf_ref[...] + jnp.where(cols == bb,
                                                  xb[:, 0:1], 0.0)

        nstep = step + NB

        @pl.when(nstep < NCH)
        def _():
            in_copy(nstep, s).start()


def _scale_kernel(x_hbm, sum_ref, sq_ref, cnt_ref, bf_ref, o_hbm,
                  inb, outb, insem, outsem, *, M, NT, BMk, NB, JK, NCH):
    i = pl.program_id(0)

    def in_copy(step, s):
        bb = step // JK
        oo = (step % JK) * BMk
        return pltpu.make_async_copy(
            x_hbm.at[bb, :, pl.ds(oo, BMk)], inb.at[s], insem.at[s])

    def out_copy(step, s):
        bb = step // JK
        oo = (step % JK) * BMk
        return pltpu.make_async_copy(
            outb.at[s], o_hbm.at[bb, :, pl.ds(oo, BMk)], outsem.at[s])

    @pl.when(i == 0)
    def _():
        for s in range(NB):
            in_copy(s, s).start()

    nrow = cnt_ref[0:1, 0:8]
    padrow = jnp.float32(M) - nrow
    bf = bf_ref[...]
    s_tot = sum_ref[:, 0:1] + jnp.sum(bf * padrow, axis=1, keepdims=True)
    q_tot = sq_ref[:, 0:1] + jnp.sum(bf * bf * padrow, axis=1, keepdims=True)
    mean = s_tot * (1.0 / NT)
    var = q_tot * (1.0 / NT) - mean * mean
    inv = jax.lax.rsqrt(var + EPS)
    lanes8 = jax.lax.broadcasted_iota(jnp.int32, (1, 8), 1)

    for s in range(NB):
        step = i * NB + s
        bb = step // JK
        jj = step % JK

        in_copy(step, s).wait()

        @pl.when(step >= NB)
        def _():
            out_copy(step - NB, s).wait()

        xb = inb[s]
        colsum = jnp.sum(xb, axis=0, keepdims=True)
        wm = colsum != 0.0

        nb_ = jnp.sum(jnp.where(lanes8 == bb, nrow, 0.0))
        lanes = jax.lax.broadcasted_iota(jnp.int32, wm.shape, 1)
        wm = wm | ((jj == 0) & (nb_ < M) & (lanes == 0))

        outb[s] = jnp.where(wm, xb * inv, xb)
        out_copy(step, s).start()

        nstep = step + NB

        @pl.when(nstep < NCH)
        def _():
            in_copy(nstep, s).start()

    @pl.when(i == (NCH // NB) - 1)
    def _():
        for s in range(NB):
            out_copy(NCH - NB + s, s).wait()


def kernel(x):
    B, C, W, H = x.shape
    M = W * H
    BM = 12544
    J = M // BM
    xr = x.reshape(B, C, M)

    stats_shapes = [
        jax.ShapeDtypeStruct((C, 128), jnp.float32),
        jax.ShapeDtypeStruct((C, 128), jnp.float32),
        jax.ShapeDtypeStruct((1, 128), jnp.float32),
        jax.ShapeDtypeStruct((C, 8), jnp.float32),
    ]
    BMs = 3584
    JKs = M // BMs
    NCHs = B * JKs
    NBs = 7

    def cs1(shape):
        return pl.BlockSpec(shape, lambda i: (0,) * len(shape))

    sums, sqs, cnts, bf = pl.pallas_call(
        functools.partial(_stats_kernel, BMk=BMs, NB=NBs, JK=JKs, NCH=NCHs),
        grid=(NCHs // NBs,),
        in_specs=[pl.BlockSpec(memory_space=pl.ANY)],
        out_specs=[cs1(s.shape) for s in stats_shapes],
        out_shape=stats_shapes,
        scratch_shapes=[
            pltpu.VMEM((NBs, C, BMs), jnp.float32),
            pltpu.SemaphoreType.DMA((NBs,)),
        ],
    )(xr)

    BMk = 3584
    JK = M // BMk
    NCH = B * JK
    NB = 7
    c_spec = pl.BlockSpec(memory_space=pl.ANY)

    def cs(shape):
        return pl.BlockSpec(shape, lambda i: (0,) * len(shape))

    out = pl.pallas_call(
        functools.partial(_scale_kernel, M=M, NT=float(B * M),
                          BMk=BMk, NB=NB, JK=JK, NCH=NCH),
        grid=(NCH // NB,),
        in_specs=[
            c_spec,
            cs((C, 128)),
            cs((C, 128)),
            cs((1, 128)),
            cs((C, 8)),
        ],
        out_specs=c_spec,
        out_shape=jax.ShapeDtypeStruct((B, C, M), jnp.float32),
        scratch_shapes=[
            pltpu.VMEM((NB, C, BMk), jnp.float32),
            pltpu.VMEM((NB, C, BMk), jnp.float32),
            pltpu.SemaphoreType.DMA((NB,)),
            pltpu.SemaphoreType.DMA((NB,)),
        ],
    )(xr, sums, sqs, cnts, bf)

    return out.reshape(B, C, W, H)
